# baseline (speedup 1.0000x reference)
import functools

import jax
import jax.numpy as jnp
from jax import lax
from jax.experimental import pallas as pl
from jax.experimental.pallas import tpu as pltpu
from jax.experimental.pallas import tpu_sc as plsc

VOCAB = 1000000
D = 64
B = 16384
NC = 2
NS = 16
L = 16
NW = NC * NS
BPW = B // NW
NG = BPW // 128
NCHK = BPW // L

_mesh = plsc.VectorSubcoreMesh(core_axis_name="c", subcore_axis_name="s")


@functools.partial(
    pl.kernel,
    mesh=_mesh,
    compiler_params=pltpu.CompilerParams(
        needs_layout_passes=False, use_tc_tiling_on_sc=False),
    out_type=jax.ShapeDtypeStruct((B,), jnp.float32),
    scratch_types=[
        pltpu.VMEM((BPW,), jnp.int32),
        pltpu.VMEM((BPW,), jnp.int32),
        pltpu.VMEM((D, BPW), jnp.float32),
        pltpu.VMEM((D, BPW), jnp.float32),
        pltpu.VMEM((BPW,), jnp.float32),
        pltpu.SemaphoreType.DMA,
        pltpu.SemaphoreType.DMA,
    ],
)
def _sg_hs_kernel(embT, cs, nodes, out, cs_i, nd_i, va, vb, o_v, sa, sb):
    wid = lax.axis_index("s") * NC + lax.axis_index("c")
    base = wid * BPW
    pltpu.sync_copy(cs.at[pl.ds(base, BPW)], cs_i)
    pltpu.sync_copy(nodes.at[pl.ds(base, BPW)], nd_i)

    def fire(f, carry):
        for g in range(NG):
            pltpu.async_copy(embT.at[f].at[cs_i.at[pl.ds(g * 128, 128)]],
                             va.at[f, pl.ds(g * 128, 128)], sa)
            pltpu.async_copy(embT.at[f].at[nd_i.at[pl.ds(g * 128, 128)]],
                             vb.at[f, pl.ds(g * 128, 128)], sb)
        return carry

    lax.fori_loop(0, D, fire, 0)
    pltpu.make_async_copy(embT.at[pl.ds(0, D), pl.ds(0, BPW)], va, sa).wait()
    pltpu.make_async_copy(embT.at[pl.ds(0, D), pl.ds(0, BPW)], vb, sb).wait()

    def compute(c, carry):
        def step(f, acc):
            av = va[f, pl.ds(c * L, L)]
            bv = vb[f, pl.ds(c * L, L)]
            return acc + av * bv

        acc = lax.fori_loop(0, D, step, jnp.zeros((L,), jnp.float32),
                            unroll=8)
        o_v[pl.ds(c * L, L)] = 1.0 / (1.0 + jnp.exp(-acc))
        return carry

    lax.fori_loop(0, NCHK, compute, 0)

    pltpu.sync_copy(o_v, out.at[pl.ds(base, BPW)])


def kernel(emb, cs, nodes):
    return _sg_hs_kernel(emb.T, cs, nodes)


# speedup vs baseline: 13.7060x; 13.7060x over previous
import functools

import jax
import jax.numpy as jnp
from jax import lax
from jax.experimental import pallas as pl
from jax.experimental.pallas import tpu as pltpu
from jax.experimental.pallas import tpu_sc as plsc

VOCAB = 1000000
D = 64
B = 16384
NC = 2
NS = 16
L = 16
NW = NC * NS
BPW = B // NW
NCHK = BPW // L

_mesh = plsc.VectorSubcoreMesh(core_axis_name="c", subcore_axis_name="s")


@functools.partial(
    pl.kernel,
    mesh=_mesh,
    compiler_params=pltpu.CompilerParams(
        needs_layout_passes=False, use_tc_tiling_on_sc=True),
    out_type=jax.ShapeDtypeStruct((B,), jnp.float32),
    scratch_types=[
        pltpu.VMEM((BPW,), jnp.int32),
        pltpu.VMEM((BPW,), jnp.int32),
        pltpu.VMEM((128, D), jnp.float32),
        pltpu.VMEM((128, D), jnp.float32),
        pltpu.VMEM((L, L), jnp.float32),
        pltpu.VMEM((BPW,), jnp.float32),
        pltpu.SemaphoreType.DMA,
        pltpu.SemaphoreType.DMA,
    ],
)
def _sg_hs_kernel(emb, cs, nodes, out, cs_i, nd_i, ar, br, t_v, o_v, sa, sb):
    wid = lax.axis_index("s") * NC + lax.axis_index("c")
    base = wid * BPW
    pltpu.sync_copy(cs.at[pl.ds(base, BPW)], cs_i)
    pltpu.sync_copy(nodes.at[pl.ds(base, BPW)], nd_i)

    rows16 = lax.iota(jnp.int32, L)

    for w in range(BPW // 128):

        def fire(c, carry, w=w):
            va = cs_i[pl.ds(c * L, L)]
            vb = nd_i[pl.ds(c * L, L)]
            for j in range(L):
                i = (c - w * 8) * L + j
                pltpu.async_copy(emb.at[pl.ds(va[j], 1)],
                                 ar.at[pl.ds(i, 1)], sa)
                pltpu.async_copy(emb.at[pl.ds(vb[j], 1)],
                                 br.at[pl.ds(i, 1)], sb)
            return carry

        lax.fori_loop(w * 8, w * 8 + 8, fire, 0)
        pltpu.make_async_copy(emb.at[pl.ds(0, 128)], ar, sa).wait()
        pltpu.make_async_copy(emb.at[pl.ds(0, 128)], br, sb).wait()

        def compute(c, carry, w=w):
            r0 = (c - w * 8) * L
            for j in range(L):
                r = r0 + j
                partial = ar[r, pl.ds(0, L)] * br[r, pl.ds(0, L)]
                for k in range(1, D // L):
                    partial = partial + (ar[r, pl.ds(k * L, L)]
                                         * br[r, pl.ds(k * L, L)])
                t_v[j, pl.ds(0, L)] = partial
            y = plsc.load_gather(t_v, [rows16, jnp.zeros((L,), jnp.int32)])
            for col in range(1, L):
                y = y + plsc.load_gather(
                    t_v, [rows16, jnp.full((L,), col, jnp.int32)])
            o_v[pl.ds(c * L, L)] = 1.0 / (1.0 + jnp.exp(-y))
            return carry

        lax.fori_loop(w * 8, w * 8 + 8, compute, 0)

    pltpu.sync_copy(o_v, out.at[pl.ds(base, BPW)])


def kernel(emb, cs, nodes):
    return _sg_hs_kernel(emb, cs, nodes)
